# Initial kernel scaffold; baseline (speedup 1.0000x reference)
import jax
import jax.numpy as jnp
from jax.experimental import pallas as pl


def kernel(x, embedding, W_lstm, U_lstm, b_lstm, W_dense, b_dense):
    raise NotImplementedError("write your pallas kernel here")


# speedup vs baseline: 7.8442x; 7.8442x over previous
import functools

import jax
import jax.numpy as jnp
from jax import lax
from jax.experimental import pallas as pl
from jax.experimental.pallas import tpu as pltpu
from jax.experimental.pallas import tpu_sc as plsc

EMB = 64
HID = 400
OUT = 1000
B, T, W = 1024, 50, 20

R = B * T
NW = 32
ROWS_PER_W = R // NW
BLK = 64
NBLK = ROWS_PER_W // BLK
LANES = 16


def _sc_gather_sum(xflat, embedding):
    mesh = plsc.VectorSubcoreMesh(core_axis_name="c", subcore_axis_name="s")

    @functools.partial(
        pl.kernel,
        out_type=jax.ShapeDtypeStruct((R, EMB), jnp.float32),
        mesh=mesh,
        scratch_types=[
            pltpu.VMEM((BLK * W,), jnp.int32),
            pltpu.VMEM((BLK * W, EMB), jnp.float32),
            pltpu.VMEM((BLK, EMB), jnp.float32),
            pltpu.SemaphoreType.DMA,
        ],
        compiler_params=pltpu.CompilerParams(use_tc_tiling_on_sc=False),
    )
    def gather_sum(x_hbm, emb_hbm, out_hbm, idx_v, rows_v, out_v, sem):
        wid = lax.axis_index("s") * 2 + lax.axis_index("c")
        base0 = wid * ROWS_PER_W

        @pl.loop(0, NBLK)
        def _(blk):
            base = base0 + blk * BLK
            pltpu.sync_copy(x_hbm.at[pl.ds(base * W, BLK * W)], idx_v)
            pltpu.async_copy(emb_hbm.at[idx_v], rows_v, sem).wait()

            @pl.loop(0, BLK)
            def _(k):
                for c in range(EMB // LANES):
                    sl = pl.ds(c * LANES, LANES)
                    acc = rows_v[k * W, sl]
                    for w in range(1, W):
                        acc = acc + rows_v[k * W + w, sl]
                    out_v[k, sl] = acc

            pltpu.sync_copy(out_v, out_hbm.at[pl.ds(base, BLK)])

    return gather_sum(xflat, embedding)


def _tc_lstm_body(s_ref, wi, wf, wg, wo, ui, uf, ug, uo,
                  bi, bf, bg, bo, wd, bd, out_ref):
    def step(t, carry):
        h, c = carry
        xt = s_ref[t]
        dot = lambda a, b: jnp.dot(a, b, preferred_element_type=jnp.float32)
        zi = dot(xt, wi[...]) + dot(h, ui[...]) + bi[...]
        zf = dot(xt, wf[...]) + dot(h, uf[...]) + bf[...]
        zg = dot(xt, wg[...]) + dot(h, ug[...]) + bg[...]
        zo = dot(xt, wo[...]) + dot(h, uo[...]) + bo[...]
        ig = jax.nn.sigmoid(zi)
        fg = jax.nn.sigmoid(zf)
        gg = jnp.tanh(zg)
        og = jax.nn.sigmoid(zo)
        c2 = fg * c + ig * gg
        h2 = og * jnp.tanh(c2)
        return (h2, c2)

    z0 = jnp.zeros((B, HID), jnp.float32)
    h, _ = lax.fori_loop(0, T, step, (z0, z0))
    logits = jnp.dot(h, wd[...], preferred_element_type=jnp.float32) + bd[...]
    m = jnp.max(logits, axis=-1, keepdims=True)
    e = jnp.exp(logits - m)
    out_ref[...] = e / jnp.sum(e, axis=-1, keepdims=True)


def _tc_lstm(s, W_lstm, U_lstm, b_lstm, W_dense, b_dense):
    ws = [W_lstm[:, i * HID:(i + 1) * HID] for i in range(4)]
    us = [U_lstm[:, i * HID:(i + 1) * HID] for i in range(4)]
    bs = [b_lstm[i * HID:(i + 1) * HID].reshape(1, HID) for i in range(4)]
    return pl.pallas_call(
        _tc_lstm_body,
        out_shape=jax.ShapeDtypeStruct((B, OUT), jnp.float32),
    )(s, *ws, *us, *bs, W_dense, b_dense.reshape(1, OUT))


def kernel(x, embedding, W_lstm, U_lstm, b_lstm, W_dense, b_dense):
    xflat = x.transpose(1, 0, 2).reshape(R * W)
    s = _sc_gather_sum(xflat, embedding)
    s = s.reshape(T, B, EMB)
    return _tc_lstm(s, W_lstm, U_lstm, b_lstm, W_dense, b_dense)


# speedup vs baseline: 10.3884x; 1.3243x over previous
import functools

import jax
import jax.numpy as jnp
from jax import lax
from jax.experimental import pallas as pl
from jax.experimental.pallas import tpu as pltpu
from jax.experimental.pallas import tpu_sc as plsc

EMB = 64
HID = 400
OUT = 1000
B, T, W = 1024, 50, 20

R = B * T
NW = 32
ROWS_PER_W = R // NW
BLK = 32
NBLK = ROWS_PER_W // BLK
LANES = 16
NCHUNK = EMB // LANES


def _sc_gather_sum(xflat, embedding):
    mesh = plsc.VectorSubcoreMesh(core_axis_name="c", subcore_axis_name="s")

    @functools.partial(
        pl.kernel,
        out_type=jax.ShapeDtypeStruct((R, EMB), jnp.float32),
        mesh=mesh,
        scratch_types=[
            pltpu.VMEM((BLK * W,), jnp.int32),
            pltpu.VMEM((BLK * W,), jnp.int32),
            pltpu.VMEM((BLK * W, EMB), jnp.float32),
            pltpu.VMEM((BLK * W, EMB), jnp.float32),
            pltpu.VMEM((BLK, EMB), jnp.float32),
            pltpu.SemaphoreType.DMA,
            pltpu.SemaphoreType.DMA,
        ],
        compiler_params=pltpu.CompilerParams(use_tc_tiling_on_sc=False),
    )
    def gather_sum(x_hbm, emb_hbm, out_hbm, idx0, idx1, rows0, rows1,
                   out_v, sem0, sem1):
        wid = lax.axis_index("s") * 2 + lax.axis_index("c")
        base0 = wid * ROWS_PER_W
        bufs = ((idx0, rows0, sem0), (idx1, rows1, sem1))

        def start(blk, buf):
            idx_v, rows_v, sem = buf
            base = base0 + blk * BLK
            pltpu.sync_copy(x_hbm.at[pl.ds(base * W, BLK * W)], idx_v)
            pltpu.make_async_copy(emb_hbm.at[idx_v], rows_v, sem).start()

        def finish(blk, buf):
            idx_v, rows_v, sem = buf
            pltpu.make_async_copy(emb_hbm.at[idx_v], rows_v, sem).wait()

            @pl.loop(0, BLK)
            def _(k):
                row = k * W
                accs = [rows_v[row, pl.ds(c * LANES, LANES)]
                        for c in range(NCHUNK)]
                for w in range(1, W):
                    accs = [accs[c] + rows_v[row + w, pl.ds(c * LANES, LANES)]
                            for c in range(NCHUNK)]
                for c in range(NCHUNK):
                    out_v[k, pl.ds(c * LANES, LANES)] = accs[c]

            base = base0 + blk * BLK
            pltpu.sync_copy(out_v, out_hbm.at[pl.ds(base, BLK)])

        start(0, bufs[0])
        start(1, bufs[1])

        @pl.loop(0, NBLK - 2, step=2)
        def _(g):
            for b in range(2):
                finish(g + b, bufs[b])
                start(g + b + 2, bufs[b])

        for b in range(2):
            finish(NBLK - 2 + b, bufs[b])

    return gather_sum(xflat, embedding)


def _tc_lstm_body(s_ref, wi, wf, wg, wo, ui, uf, ug, uo,
                  bi, bf, bg, bo, wd, bd, out_ref):
    def step(t, carry):
        h, c = carry
        xt = s_ref[t].astype(jnp.bfloat16)
        hb = h.astype(jnp.bfloat16)
        dot = lambda a, b: jnp.dot(a, b, preferred_element_type=jnp.float32)
        zi = dot(xt, wi[...]) + dot(hb, ui[...]) + bi[...]
        zf = dot(xt, wf[...]) + dot(hb, uf[...]) + bf[...]
        zg = dot(xt, wg[...]) + dot(hb, ug[...]) + bg[...]
        zo = dot(xt, wo[...]) + dot(hb, uo[...]) + bo[...]
        ig = jax.nn.sigmoid(zi)
        fg = jax.nn.sigmoid(zf)
        gg = jnp.tanh(zg)
        og = jax.nn.sigmoid(zo)
        c2 = fg * c + ig * gg
        h2 = og * jnp.tanh(c2)
        return (h2, c2)

    z0 = jnp.zeros((B, HID), jnp.float32)
    h, _ = lax.fori_loop(0, T, step, (z0, z0))
    logits = jnp.dot(h, wd[...], preferred_element_type=jnp.float32) + bd[...]
    m = jnp.max(logits, axis=-1, keepdims=True)
    e = jnp.exp(logits - m)
    out_ref[...] = e / jnp.sum(e, axis=-1, keepdims=True)


def _tc_lstm(s, W_lstm, U_lstm, b_lstm, W_dense, b_dense):
    wl = W_lstm.astype(jnp.bfloat16)
    ul = U_lstm.astype(jnp.bfloat16)
    ws = [wl[:, i * HID:(i + 1) * HID] for i in range(4)]
    us = [ul[:, i * HID:(i + 1) * HID] for i in range(4)]
    bs = [b_lstm[i * HID:(i + 1) * HID].reshape(1, HID) for i in range(4)]
    return pl.pallas_call(
        _tc_lstm_body,
        out_shape=jax.ShapeDtypeStruct((B, OUT), jnp.float32),
    )(s, *ws, *us, *bs, W_dense, b_dense.reshape(1, OUT))


def kernel(x, embedding, W_lstm, U_lstm, b_lstm, W_dense, b_dense):
    xflat = x.transpose(1, 0, 2).reshape(R * W)
    s = _sc_gather_sum(xflat, embedding)
    s = s.reshape(T, B, EMB)
    return _tc_lstm(s, W_lstm, U_lstm, b_lstm, W_dense, b_dense)
